# baseline (speedup 1.0000x reference)
import functools

import jax
import jax.numpy as jnp
from jax import lax
from jax.experimental import pallas as pl
from jax.experimental.pallas import tpu as pltpu
from jax.experimental.pallas import tpu_sc as plsc

_LANES = 16
_CHUNK = 64


def _gmf_call(B, D, n_workers, v2, h2, virus_w, human_w, vb_w, hb_w, bias16):
    per_w = B // n_workers
    n_chunks = per_w // _CHUNK
    groups_per_chunk = _CHUNK // _LANES
    mesh = plsc.VectorSubcoreMesh(core_axis_name="c", subcore_axis_name="s")

    @functools.partial(
        pl.kernel,
        mesh=mesh,
        out_type=jax.ShapeDtypeStruct((B,), jnp.float32),
        scratch_types=[
            pltpu.VMEM((per_w,), jnp.int32),
            pltpu.VMEM((per_w,), jnp.int32),
            pltpu.VMEM((_CHUNK, D), jnp.float32),
            pltpu.VMEM((_CHUNK, D), jnp.float32),
            pltpu.VMEM((_CHUNK, D), jnp.float32),
            pltpu.VMEM((_CHUNK, D), jnp.float32),
            pltpu.VMEM((_CHUNK, 1), jnp.float32),
            pltpu.VMEM((_CHUNK, 1), jnp.float32),
            pltpu.VMEM((_CHUNK, 1), jnp.float32),
            pltpu.VMEM((_CHUNK, 1), jnp.float32),
            pltpu.VMEM((_LANES,), jnp.float32),
            pltpu.VMEM((per_w,), jnp.float32),
            pltpu.SemaphoreType.DMA,
            pltpu.SemaphoreType.DMA,
        ],
        compiler_params=pltpu.CompilerParams(needs_layout_passes=False),
    )
    def body(v_hbm, h_hbm, vw_hbm, hw_hbm, vb_hbm, hb_hbm, bias_hbm, out_hbm,
             vidx, hidx, u_a, v_a, u_b, v_b, bu_a, bv_a, bu_b, bv_b,
             bias_v, out_v, sem_a, sem_b):
        num_c = lax.axis_size("c")
        wid = lax.axis_index("s") * num_c + lax.axis_index("c")

        pltpu.sync_copy(v_hbm.at[wid], vidx)
        pltpu.sync_copy(h_hbm.at[wid], hidx)
        pltpu.sync_copy(bias_hbm, bias_v)

        iota = lax.iota(jnp.int32, _LANES)

        def fire_chunk(c, u_buf, v_buf, bu_buf, bv_buf, sem):
            def issue(g, carry):
                base = c * _CHUNK + g * _LANES
                vv = vidx[pl.ds(base, _LANES)]
                hh = hidx[pl.ds(base, _LANES)]
                for l in range(_LANES):
                    j = g * _LANES + l
                    sv = vv[l]
                    sh = hh[l]
                    pltpu.async_copy(vw_hbm.at[pl.ds(sv, 1), :],
                                     u_buf.at[pl.ds(j, 1), :], sem)
                    pltpu.async_copy(hw_hbm.at[pl.ds(sh, 1), :],
                                     v_buf.at[pl.ds(j, 1), :], sem)
                    pltpu.async_copy(vb_hbm.at[pl.ds(sv, 1), :],
                                     bu_buf.at[pl.ds(j, 1), :], sem)
                    pltpu.async_copy(hb_hbm.at[pl.ds(sh, 1), :],
                                     bv_buf.at[pl.ds(j, 1), :], sem)
                return carry
            lax.fori_loop(0, groups_per_chunk, issue, 0)

        def drain_chunk(u_buf, v_buf, bu_buf, bv_buf, sem):
            pltpu.make_async_copy(vw_hbm.at[pl.ds(0, _CHUNK), :],
                                  u_buf, sem).wait()
            pltpu.make_async_copy(hw_hbm.at[pl.ds(0, _CHUNK), :],
                                  v_buf, sem).wait()
            pltpu.make_async_copy(vb_hbm.at[pl.ds(0, _CHUNK), :],
                                  bu_buf, sem).wait()
            pltpu.make_async_copy(hb_hbm.at[pl.ds(0, _CHUNK), :],
                                  bv_buf, sem).wait()

        def compute_chunk(c, u_buf, v_buf, bu_buf, bv_buf):
            bias_vec = bias_v[...]

            def group(g, carry):
                row = g * _LANES + iota
                zeros = iota * 0
                acc = (bias_vec
                       + plsc.load_gather(bu_buf, [row, zeros])
                       + plsc.load_gather(bv_buf, [row, zeros]))
                for d in range(D):
                    col = jnp.full((_LANES,), d, jnp.int32)
                    ug = plsc.load_gather(u_buf, [row, col])
                    vg = plsc.load_gather(v_buf, [row, col])
                    acc = acc + ug * vg
                out_v[pl.ds(c * _CHUNK + g * _LANES, _LANES)] = acc
                return carry
            lax.fori_loop(0, groups_per_chunk, group, 0)

        bufs = ((u_a, v_a, bu_a, bv_a, sem_a), (u_b, v_b, bu_b, bv_b, sem_b))
        fire_chunk(0, *bufs[0])
        if n_chunks > 1:
            fire_chunk(1, *bufs[1])
        for c in range(n_chunks):
            u_buf, v_buf, bu_buf, bv_buf, sem = bufs[c % 2]
            drain_chunk(u_buf, v_buf, bu_buf, bv_buf, sem)
            compute_chunk(c, u_buf, v_buf, bu_buf, bv_buf)
            if c + 2 < n_chunks:
                fire_chunk(c + 2, u_buf, v_buf, bu_buf, bv_buf, sem)

        pltpu.sync_copy(out_v, out_hbm.at[pl.ds(wid * per_w, per_w)])

    return body(v2, h2, virus_w, human_w, vb_w, hb_w, bias16)


def kernel(v_idxs, h_idxs, virus_w, human_w, vb_w, hb_w, bias):
    B = v_idxs.shape[0]
    D = virus_w.shape[1]
    info = plsc.get_sparse_core_info()
    n_workers = info.num_cores * info.num_subcores
    v2 = v_idxs.astype(jnp.int32).reshape(n_workers, B // n_workers)
    h2 = h_idxs.astype(jnp.int32).reshape(n_workers, B // n_workers)
    bias16 = jnp.broadcast_to(bias.astype(jnp.float32), (_LANES,))
    return _gmf_call(B, D, n_workers, v2, h2, virus_w, human_w,
                     vb_w, hb_w, bias16)


# speedup vs baseline: 1.2078x; 1.2078x over previous
import functools

import jax
import jax.numpy as jnp
from jax import lax
from jax.experimental import pallas as pl
from jax.experimental.pallas import tpu as pltpu
from jax.experimental.pallas import tpu_sc as plsc

_LANES = 16
_CHUNK = 128


def _gmf_call(B, D, n_workers, v3, h3, v83, h83, r8v, r8h,
              vt, ht, vb, hb, bias16):
    per_w = B // n_workers
    n_chunks = per_w // _CHUNK
    groups_per_chunk = _CHUNK // _LANES
    sup_w = 128 // D
    mesh = plsc.VectorSubcoreMesh(core_axis_name="c", subcore_axis_name="s")

    @functools.partial(
        pl.kernel,
        mesh=mesh,
        out_type=jax.ShapeDtypeStruct((B,), jnp.float32),
        scratch_types=[
            pltpu.VMEM((n_chunks, _CHUNK), jnp.int32),
            pltpu.VMEM((n_chunks, _CHUNK), jnp.int32),
            pltpu.VMEM((n_chunks, _CHUNK), jnp.int32),
            pltpu.VMEM((n_chunks, _CHUNK), jnp.int32),
            pltpu.VMEM((per_w,), jnp.int32),
            pltpu.VMEM((per_w,), jnp.int32),
            pltpu.VMEM((_CHUNK, 128), jnp.float32),
            pltpu.VMEM((_CHUNK, 128), jnp.float32),
            pltpu.VMEM((_CHUNK, 128), jnp.float32),
            pltpu.VMEM((_CHUNK, 128), jnp.float32),
            pltpu.VMEM((per_w,), jnp.float32),
            pltpu.VMEM((per_w,), jnp.float32),
            pltpu.VMEM((_LANES,), jnp.float32),
            pltpu.VMEM((per_w,), jnp.float32),
            pltpu.SemaphoreType.DMA,
            pltpu.SemaphoreType.DMA,
            pltpu.SemaphoreType.DMA,
        ],
        compiler_params=pltpu.CompilerParams(needs_layout_passes=False),
    )
    def body(v_hbm, h_hbm, v8_hbm, h8_hbm, r8v_hbm, r8h_hbm,
             vt_hbm, ht_hbm, vb_hbm, hb_hbm, bias_hbm, out_hbm,
             vidx, hidx, v8idx, h8idx, vcol, hcol,
             u_a, v_a, u_b, v_b, bu, bv, bias_v, out_v,
             sem_a, sem_b, sem_c):
        num_c = lax.axis_size("c")
        wid = lax.axis_index("s") * num_c + lax.axis_index("c")

        pltpu.sync_copy(v_hbm.at[wid], vidx)
        pltpu.sync_copy(h_hbm.at[wid], hidx)
        pltpu.sync_copy(v8_hbm.at[wid], v8idx)
        pltpu.sync_copy(h8_hbm.at[wid], h8idx)
        pltpu.sync_copy(r8v_hbm.at[wid], vcol)
        pltpu.sync_copy(r8h_hbm.at[wid], hcol)
        pltpu.sync_copy(bias_hbm, bias_v)

        bias_copies = []
        for c in range(n_chunks):
            rows = pl.ds(c * _CHUNK, _CHUNK)
            bias_copies.append(
                pltpu.async_copy(vb_hbm.at[vidx.at[c]], bu.at[rows], sem_c))
            bias_copies.append(
                pltpu.async_copy(hb_hbm.at[hidx.at[c]], bv.at[rows], sem_c))

        def fire(c, u_buf, v_buf, sem):
            return (
                pltpu.async_copy(vt_hbm.at[v8idx.at[c]], u_buf, sem),
                pltpu.async_copy(ht_hbm.at[h8idx.at[c]], v_buf, sem),
            )

        iota = lax.iota(jnp.int32, _LANES)

        def compute_chunk(c, u_buf, v_buf):
            bias_vec = bias_v[...]

            def group(g, carry):
                i0 = c * _CHUNK + g * _LANES
                row = g * _LANES + iota
                cu = vcol[pl.ds(i0, _LANES)]
                ch = hcol[pl.ds(i0, _LANES)]
                lanes = pl.ds(i0, _LANES)
                acc = bias_vec + bu[lanes] + bv[lanes]
                for d in range(D):
                    ug = plsc.load_gather(u_buf, [row, cu + d])
                    vg = plsc.load_gather(v_buf, [row, ch + d])
                    acc = acc + ug * vg
                out_v[pl.ds(i0, _LANES)] = acc
                return carry
            lax.fori_loop(0, groups_per_chunk, group, 0)

        bufs = ((u_a, v_a, sem_a), (u_b, v_b, sem_b))
        pending = {}
        pending[0] = fire(0, *bufs[0])
        if n_chunks > 1:
            pending[1] = fire(1, *bufs[1])
        for cp in bias_copies:
            cp.wait()
        for c in range(n_chunks):
            u_buf, v_buf, _ = bufs[c % 2]
            for cp in pending.pop(c):
                cp.wait()
            compute_chunk(c, u_buf, v_buf)
            if c + 2 < n_chunks:
                pending[c + 2] = fire(c + 2, *bufs[c % 2])

        pltpu.sync_copy(out_v, out_hbm.at[pl.ds(wid * per_w, per_w)])

    return body(v3, h3, v83, h83, r8v, r8h, vt, ht, vb, hb, bias16)


def kernel(v_idxs, h_idxs, virus_w, human_w, vb_w, hb_w, bias):
    B = v_idxs.shape[0]
    D = virus_w.shape[1]
    sup_w = 128 // D
    info = plsc.get_sparse_core_info()
    n_workers = info.num_cores * info.num_subcores
    n_chunks = B // n_workers // _CHUNK
    vi = v_idxs.astype(jnp.int32)
    hi = h_idxs.astype(jnp.int32)
    shp = (n_workers, n_chunks, _CHUNK)
    v3 = vi.reshape(shp)
    h3 = hi.reshape(shp)
    v83 = (vi // sup_w).reshape(shp)
    h83 = (hi // sup_w).reshape(shp)
    r8v = ((vi % sup_w) * D).reshape(n_workers, -1)
    r8h = ((hi % sup_w) * D).reshape(n_workers, -1)
    vt = virus_w.reshape(-1, 128)
    ht = human_w.reshape(-1, 128)
    bias16 = jnp.broadcast_to(bias.astype(jnp.float32), (_LANES,))
    return _gmf_call(B, D, n_workers, v3, h3, v83, h83, r8v, r8h,
                     vt, ht, vb_w.reshape(-1), hb_w.reshape(-1), bias16)


# speedup vs baseline: 1.2176x; 1.0081x over previous
import functools

import jax
import jax.numpy as jnp
from jax import lax
from jax.experimental import pallas as pl
from jax.experimental.pallas import tpu as pltpu
from jax.experimental.pallas import tpu_sc as plsc

_LANES = 16
_CHUNK = 128


def _gmf_call(B, D, n_workers, v3, h3, virus_w, human_w, vb_w, hb_w, bias16):
    per_w = B // n_workers
    n_chunks = per_w // _CHUNK
    n_groups = per_w // _LANES
    mesh = plsc.VectorSubcoreMesh(core_axis_name="c", subcore_axis_name="s")

    @functools.partial(
        pl.kernel,
        mesh=mesh,
        out_type=jax.ShapeDtypeStruct((B,), jnp.float32),
        scratch_types=[
            pltpu.VMEM((n_chunks, _CHUNK), jnp.int32),
            pltpu.VMEM((n_chunks, _CHUNK), jnp.int32),
            pltpu.VMEM((per_w, D), jnp.float32),
            pltpu.VMEM((per_w, D), jnp.float32),
            pltpu.VMEM((per_w,), jnp.float32),
            pltpu.VMEM((per_w,), jnp.float32),
            pltpu.VMEM((_LANES,), jnp.float32),
            pltpu.VMEM((per_w,), jnp.float32),
            pltpu.SemaphoreType.DMA,
        ],
        compiler_params=pltpu.CompilerParams(
            needs_layout_passes=False, use_tc_tiling_on_sc=False),
    )
    def body(v_hbm, h_hbm, vw_hbm, hw_hbm, vb_hbm, hb_hbm, bias_hbm, out_hbm,
             vidx, hidx, u_rows, v_rows, bu, bv, bias_v, out_v, sem):
        num_c = lax.axis_size("c")
        wid = lax.axis_index("s") * num_c + lax.axis_index("c")

        pltpu.sync_copy(v_hbm.at[wid], vidx)
        pltpu.sync_copy(h_hbm.at[wid], hidx)
        pltpu.sync_copy(bias_hbm, bias_v)

        u2d = u_rows
        v2d = v_rows
        copies = []
        for j in range(n_chunks):
            rows = pl.ds(j * _CHUNK, _CHUNK)
            copies.append(pltpu.async_copy(vw_hbm.at[vidx.at[j]],
                                           u2d.at[rows], sem))
            copies.append(pltpu.async_copy(hw_hbm.at[hidx.at[j]],
                                           v2d.at[rows], sem))
            copies.append(pltpu.async_copy(vb_hbm.at[vidx.at[j]],
                                           bu.at[rows], sem))
            copies.append(pltpu.async_copy(hb_hbm.at[hidx.at[j]],
                                           bv.at[rows], sem))
        for cp in copies:
            cp.wait()

        iota = lax.iota(jnp.int32, _LANES)
        bias_vec = bias_v[...]
        u_flat = u_rows
        v_flat = v_rows

        def group(g, carry):
            row = g * _LANES + iota
            lanes = pl.ds(g * _LANES, _LANES)
            acc = bias_vec + bu[lanes] + bv[lanes]
            for d in range(D):
                col = jnp.full((_LANES,), d, jnp.int32)
                ug = plsc.load_gather(u2d, [row, col])
                vg = plsc.load_gather(v2d, [row, col])
                acc = acc + ug * vg
            out_v[pl.ds(g * _LANES, _LANES)] = acc
            return carry

        lax.fori_loop(0, n_groups, group, 0)
        pltpu.sync_copy(out_v, out_hbm.at[pl.ds(wid * per_w, per_w)])

    return body(v3, h3, virus_w, human_w, vb_w, hb_w, bias16)


def kernel(v_idxs, h_idxs, virus_w, human_w, vb_w, hb_w, bias):
    B = v_idxs.shape[0]
    D = virus_w.shape[1]
    info = plsc.get_sparse_core_info()
    n_workers = info.num_cores * info.num_subcores
    n_chunks = B // n_workers // _CHUNK
    v3 = v_idxs.astype(jnp.int32).reshape(n_workers, n_chunks, _CHUNK)
    h3 = h_idxs.astype(jnp.int32).reshape(n_workers, n_chunks, _CHUNK)
    bias16 = jnp.broadcast_to(bias.astype(jnp.float32), (_LANES,))
    return _gmf_call(B, D, n_workers, v3, h3, virus_w, human_w,
                     vb_w.reshape(-1), hb_w.reshape(-1), bias16)


# speedup vs baseline: 1.2195x; 1.0016x over previous
import functools

import jax
import jax.numpy as jnp
from jax import lax
from jax.experimental import pallas as pl
from jax.experimental.pallas import tpu as pltpu
from jax.experimental.pallas import tpu_sc as plsc

_LANES = 16
_CHUNK = 128


def _gmf_call(B, D, n_workers, v3, h3, virus_w, human_w, vb_w, hb_w, bias16):
    per_w = B // n_workers
    n_chunks = per_w // _CHUNK
    n_groups = per_w // _LANES
    mesh = plsc.VectorSubcoreMesh(core_axis_name="c", subcore_axis_name="s")

    @functools.partial(
        pl.kernel,
        mesh=mesh,
        out_type=jax.ShapeDtypeStruct((B,), jnp.float32),
        scratch_types=[
            pltpu.VMEM((n_chunks, _CHUNK), jnp.int32),
            pltpu.VMEM((n_chunks, _CHUNK), jnp.int32),
            pltpu.VMEM((per_w, D), jnp.float32),
            pltpu.VMEM((per_w, D), jnp.float32),
            pltpu.VMEM((per_w,), jnp.float32),
            pltpu.VMEM((per_w,), jnp.float32),
            pltpu.VMEM((_LANES,), jnp.float32),
            pltpu.VMEM((per_w,), jnp.float32),
            pltpu.SemaphoreType.DMA,
        ],
        compiler_params=pltpu.CompilerParams(
            needs_layout_passes=False, use_tc_tiling_on_sc=False),
    )
    def body(v_hbm, h_hbm, vw_hbm, hw_hbm, vb_hbm, hb_hbm, bias_hbm, out_hbm,
             vidx, hidx, u_rows, v_rows, bu, bv, bias_v, out_v, sem):
        num_c = lax.axis_size("c")
        wid = lax.axis_index("s") * num_c + lax.axis_index("c")

        pltpu.sync_copy(v_hbm.at[wid], vidx)
        pltpu.sync_copy(h_hbm.at[wid], hidx)
        pltpu.sync_copy(bias_hbm, bias_v)

        copies = []
        for j in range(n_chunks):
            rows = pl.ds(j * _CHUNK, _CHUNK)
            copies.append(pltpu.async_copy(vw_hbm.at[vidx.at[j]],
                                           u_rows.at[rows], sem))
            copies.append(pltpu.async_copy(hw_hbm.at[hidx.at[j]],
                                           v_rows.at[rows], sem))
            copies.append(pltpu.async_copy(vb_hbm.at[vidx.at[j]],
                                           bu.at[rows], sem))
            copies.append(pltpu.async_copy(hb_hbm.at[hidx.at[j]],
                                           bv.at[rows], sem))
        for cp in copies:
            cp.wait()

        iota = lax.iota(jnp.int32, _LANES)
        bias_vec = bias_v[...]

        def group(g, carry):
            row = g * _LANES + iota
            lanes = pl.ds(g * _LANES, _LANES)
            acc = bias_vec + bu[lanes] + bv[lanes]
            for d in range(D):
                col = jnp.full((_LANES,), d, jnp.int32)
                ug = plsc.load_gather(u_rows, [row, col])
                vg = plsc.load_gather(v_rows, [row, col])
                acc = acc + ug * vg
            out_v[pl.ds(g * _LANES, _LANES)] = acc
            return carry

        lax.fori_loop(0, n_groups, group, 0)
        pltpu.sync_copy(out_v, out_hbm.at[pl.ds(wid * per_w, per_w)])

    return body(v3, h3, virus_w, human_w, vb_w, hb_w, bias16)


def kernel(v_idxs, h_idxs, virus_w, human_w, vb_w, hb_w, bias):
    B = v_idxs.shape[0]
    D = virus_w.shape[1]
    info = plsc.get_sparse_core_info()
    n_workers = info.num_cores * info.num_subcores
    n_chunks = B // n_workers // _CHUNK
    v3 = v_idxs.astype(jnp.int32).reshape(n_workers, n_chunks, _CHUNK)
    h3 = h_idxs.astype(jnp.int32).reshape(n_workers, n_chunks, _CHUNK)
    bias16 = jnp.broadcast_to(bias.astype(jnp.float32), (_LANES,))
    return _gmf_call(B, D, n_workers, v3, h3, virus_w, human_w,
                     vb_w.reshape(-1), hb_w.reshape(-1), bias16)
